# baseline (speedup 1.0000x reference)
import functools

import jax
import jax.numpy as jnp
from jax import lax
from jax.experimental import pallas as pl
from jax.experimental.pallas import tpu as pltpu
from jax.experimental.pallas import tpu_sc as plsc

_REMAP = (1, 2, 3, 4, 5, 6, 7, 8, 9, 10, 11, 13, 14, 15, 16, 17, 18, 19,
          20, 21, 22, 23, 24, 25, 27, 28, 31, 32, 33, 34, 35, 36, 37, 38,
          39, 40, 41, 42, 43, 44, 46, 47, 48, 49, 50, 51, 52, 53, 54, 55,
          56, 57, 58, 59, 60, 61, 62, 63, 64, 65, 67, 70, 72, 73, 74, 75,
          76, 77, 78, 79, 80, 81, 82, 84, 85, 86, 87, 88, 89, 90)

_NC, _NS, _L = 2, 16, 16
_NW = _NC * _NS
_R = 160
_IN_W = _R * 91
_OUT_W = _R * 80
_NV = _OUT_W // _L

_ROWS_PRED = 16 * 900
_ROWS_AUX = 6 * 16 * 900
_CH_PRED = _ROWS_PRED // _R
_CH_AUX = _ROWS_AUX // _R


def _sc_body(remap_hbm, pred_in, enc_in, aux_in, pred_out, enc_out, aux_out,
             in_v, out_v, idx_v, rem_v):
    wid = lax.axis_index("s") * _NC + lax.axis_index("c")

    pltpu.sync_copy(remap_hbm, rem_v)

    def build(r, carry):
        base = r * 80
        off = lax.broadcast(r * 91, (_L,))
        for j in range(5):
            idx_v[pl.ds(base + j * _L, _L)] = rem_v[pl.ds(j * _L, _L)] + off
        return carry

    lax.fori_loop(0, _R, build, 0)

    for src, dst, nchunks in ((pred_in, pred_out, _CH_PRED),
                              (enc_in, enc_out, _CH_PRED),
                              (aux_in, aux_out, _CH_AUX)):
        n_mine = (nchunks - wid + _NW - 1) // _NW

        def chunk_body(i, carry, src=src, dst=dst):
            g = wid + i * _NW
            pltpu.sync_copy(src.at[pl.ds(g * _IN_W, _IN_W)], in_v)

            def gath(t, c):
                iv = idx_v[pl.ds(t * _L, _L)]
                out_v[pl.ds(t * _L, _L)] = plsc.load_gather(in_v, [iv])
                return c

            lax.fori_loop(0, _NV, gath, 0)
            pltpu.sync_copy(out_v, dst.at[pl.ds(g * _OUT_W, _OUT_W)])
            return carry

        lax.fori_loop(0, n_mine, chunk_body, 0)


@jax.jit
def kernel(pred_logits, enc_pred_logits, aux_pred_logits):
    mesh = plsc.VectorSubcoreMesh(core_axis_name="c", subcore_axis_name="s",
                                  num_cores=_NC, num_subcores=_NS)
    run = pl.kernel(
        _sc_body,
        out_type=(
            jax.ShapeDtypeStruct((_ROWS_PRED * 80,), jnp.float32),
            jax.ShapeDtypeStruct((_ROWS_PRED * 80,), jnp.float32),
            jax.ShapeDtypeStruct((_ROWS_AUX * 80,), jnp.float32),
        ),
        mesh=mesh,
        scratch_types=[
            pltpu.VMEM((_IN_W,), jnp.float32),
            pltpu.VMEM((_OUT_W,), jnp.float32),
            pltpu.VMEM((_OUT_W,), jnp.int32),
            pltpu.VMEM((80,), jnp.int32),
        ],
        compiler_params=pltpu.CompilerParams(needs_layout_passes=False),
    )
    remap_arr = jnp.array(_REMAP, dtype=jnp.int32)
    out, enc_out, aux_out = run(remap_arr,
                                pred_logits.reshape(-1),
                                enc_pred_logits.reshape(-1),
                                aux_pred_logits.reshape(-1))
    return (out.reshape(16, 900, 80),
            enc_out.reshape(16, 900, 80),
            aux_out.reshape(6, 16, 900, 80))


# speedup vs baseline: 1.1501x; 1.1501x over previous
import functools

import jax
import jax.numpy as jnp
from jax import lax
from jax.experimental import pallas as pl
from jax.experimental.pallas import tpu as pltpu
from jax.experimental.pallas import tpu_sc as plsc

_REMAP = (1, 2, 3, 4, 5, 6, 7, 8, 9, 10, 11, 13, 14, 15, 16, 17, 18, 19,
          20, 21, 22, 23, 24, 25, 27, 28, 31, 32, 33, 34, 35, 36, 37, 38,
          39, 40, 41, 42, 43, 44, 46, 47, 48, 49, 50, 51, 52, 53, 54, 55,
          56, 57, 58, 59, 60, 61, 62, 63, 64, 65, 67, 70, 72, 73, 74, 75,
          76, 77, 78, 79, 80, 81, 82, 84, 85, 86, 87, 88, 89, 90)

_NC, _NS, _L = 2, 16, 16
_NW = _NC * _NS
_R = 160
_IN_W = _R * 91
_OUT_W = _R * 80
_NV = _OUT_W // _L

_ROWS_PRED = 16 * 900
_ROWS_AUX = 6 * 16 * 900
_CH_PRED = _ROWS_PRED // _R
_CH_AUX = _ROWS_AUX // _R


def _sc_body(remap_hbm, pred_in, enc_in, aux_in, pred_out, enc_out, aux_out,
             in_v, out_v, rem_v):
    wid = lax.axis_index("s") * _NC + lax.axis_index("c")

    pltpu.sync_copy(remap_hbm, rem_v)
    pats = [rem_v[pl.ds(j * _L, _L)] for j in range(5)]

    for src, dst, nchunks in ((pred_in, pred_out, _CH_PRED),
                              (enc_in, enc_out, _CH_PRED),
                              (aux_in, aux_out, _CH_AUX)):
        n_mine = (nchunks - wid + _NW - 1) // _NW

        def chunk_body(i, carry, src=src, dst=dst):
            g = wid + i * _NW
            pltpu.sync_copy(src.at[pl.ds(g * _IN_W, _IN_W)], in_v)

            @plsc.parallel_loop(0, _R, step=1, unroll=8)
            def _gath(r):
                base = r * 80
                off = lax.broadcast(r * 91, (_L,))
                for j in range(5):
                    out_v[pl.ds(base + j * _L, _L)] = plsc.load_gather(
                        in_v, [pats[j] + off])

            pltpu.sync_copy(out_v, dst.at[pl.ds(g * _OUT_W, _OUT_W)])
            return carry

        lax.fori_loop(0, n_mine, chunk_body, 0)


@jax.jit
def kernel(pred_logits, enc_pred_logits, aux_pred_logits):
    mesh = plsc.VectorSubcoreMesh(core_axis_name="c", subcore_axis_name="s",
                                  num_cores=_NC, num_subcores=_NS)
    run = pl.kernel(
        _sc_body,
        out_type=(
            jax.ShapeDtypeStruct((_ROWS_PRED * 80,), jnp.float32),
            jax.ShapeDtypeStruct((_ROWS_PRED * 80,), jnp.float32),
            jax.ShapeDtypeStruct((_ROWS_AUX * 80,), jnp.float32),
        ),
        mesh=mesh,
        scratch_types=[
            pltpu.VMEM((_IN_W,), jnp.float32),
            pltpu.VMEM((_OUT_W,), jnp.float32),
            pltpu.VMEM((80,), jnp.int32),
        ],
        compiler_params=pltpu.CompilerParams(needs_layout_passes=False),
    )
    remap_arr = jnp.array(_REMAP, dtype=jnp.int32)
    out, enc_out, aux_out = run(remap_arr,
                                pred_logits.reshape(-1),
                                enc_pred_logits.reshape(-1),
                                aux_pred_logits.reshape(-1))
    return (out.reshape(16, 900, 80),
            enc_out.reshape(16, 900, 80),
            aux_out.reshape(6, 16, 900, 80))


# speedup vs baseline: 3.9667x; 3.4491x over previous
import jax
import jax.numpy as jnp
from jax import lax
from jax.experimental import pallas as pl
from jax.experimental.pallas import tpu as pltpu
from jax.experimental.pallas import tpu_sc as plsc

_REMAP = (1, 2, 3, 4, 5, 6, 7, 8, 9, 10, 11, 13, 14, 15, 16, 17, 18, 19,
          20, 21, 22, 23, 24, 25, 27, 28, 31, 32, 33, 34, 35, 36, 37, 38,
          39, 40, 41, 42, 43, 44, 46, 47, 48, 49, 50, 51, 52, 53, 54, 55,
          56, 57, 58, 59, 60, 61, 62, 63, 64, 65, 67, 70, 72, 73, 74, 75,
          76, 77, 78, 79, 80, 81, 82, 84, 85, 86, 87, 88, 89, 90)

_NC, _NS, _L = 2, 16, 16
_NW = _NC * _NS
_RA = 184
_RB = 900 - 4 * _RA


def _sc_body(remap_hbm, pred_in, enc_in, aux_in, pred_out, enc_out, aux_out,
             in_a, out_a, in_b, out_b, rem_v):
    wid = lax.axis_index("s") * _NC + lax.axis_index("c")

    pltpu.sync_copy(remap_hbm, rem_v)
    pats = [rem_v[pl.ds(j * _L, _L)] for j in range(5)]

    def remap_chunk(src_v, dst_v, nrows):
        @plsc.parallel_loop(0, nrows, step=1, unroll=8)
        def _gath(r):
            rows = lax.broadcast(r, (_L,))
            for j in range(5):
                dst_v[r, pl.ds(j * _L, _L)] = plsc.load_gather(
                    src_v, [rows, pats[j]])

    def slab_ref(ref, slab):
        if ref.shape[:1] == (6,):
            return ref.at[slab // 16, slab % 16]
        return ref.at[slab]

    for src, dst, n_slabs in ((pred_in, pred_out, 16),
                              (enc_in, enc_out, 16),
                              (aux_in, aux_out, 96)):
        n_mine = (n_slabs * 4 - wid + _NW - 1) // _NW

        def item_body(i, carry, src=src, dst=dst):
            item = wid + i * _NW
            slab = item // 4
            q0 = pl.multiple_of((item % 4) * _RA, 8)
            pltpu.sync_copy(slab_ref(src, slab).at[pl.ds(q0, _RA), :], in_a)
            remap_chunk(in_a, out_a, _RA)
            pltpu.sync_copy(out_a, slab_ref(dst, slab).at[pl.ds(q0, _RA), :])
            return carry

        lax.fori_loop(0, n_mine, item_body, 0)

    for src, dst, n_slabs in ((pred_in, pred_out, 16),
                              (enc_in, enc_out, 16),
                              (aux_in, aux_out, 96)):
        n_mine = (n_slabs - wid + _NW - 1) // _NW

        def tail_body(i, carry, src=src, dst=dst):
            slab = wid + i * _NW
            pltpu.sync_copy(slab_ref(src, slab).at[pl.ds(4 * _RA, _RB), :],
                            in_b)
            remap_chunk(in_b, out_b, _RB)
            pltpu.sync_copy(out_b,
                            slab_ref(dst, slab).at[pl.ds(4 * _RA, _RB), :])
            return carry

        lax.fori_loop(0, n_mine, tail_body, 0)


@jax.jit
def kernel(pred_logits, enc_pred_logits, aux_pred_logits):
    mesh = plsc.VectorSubcoreMesh(core_axis_name="c", subcore_axis_name="s",
                                  num_cores=_NC, num_subcores=_NS)
    run = pl.kernel(
        _sc_body,
        out_type=(
            jax.ShapeDtypeStruct((16, 900, 80), jnp.float32),
            jax.ShapeDtypeStruct((16, 900, 80), jnp.float32),
            jax.ShapeDtypeStruct((6, 16, 900, 80), jnp.float32),
        ),
        mesh=mesh,
        scratch_types=[
            pltpu.VMEM((_RA, 91), jnp.float32),
            pltpu.VMEM((_RA, 80), jnp.float32),
            pltpu.VMEM((_RB, 91), jnp.float32),
            pltpu.VMEM((_RB, 80), jnp.float32),
            pltpu.VMEM((80,), jnp.int32),
        ],
        compiler_params=pltpu.CompilerParams(needs_layout_passes=False),
    )
    remap_arr = jnp.array(_REMAP, dtype=jnp.int32)
    return run(remap_arr, pred_logits, enc_pred_logits, aux_pred_logits)
